# baseline (speedup 1.0000x reference)
import functools

import jax
import jax.numpy as jnp
from jax import lax
from jax.experimental import pallas as pl
from jax.experimental.pallas import tpu as pltpu
from jax.experimental.pallas import tpu_sc as plsc

_N, _D, _K = 262144, 64, 512
_BLK = 2048
_NB = _N // _BLK

_NC, _NS, _L = 2, 16, 16
_NW = _NC * _NS
_BPW = _N // _NW
_CH = 256
_NCH = _BPW // _CH


def _tc_argmin_block(x_ref, emb_ref, idx_ref, s1_ref, enorm_ref):
    i = pl.program_id(0)
    x = x_ref[...]
    emb = emb_ref[...]

    @pl.when(i == 0)
    def _():
        enorm_ref[...] = jnp.sum(emb * emb, axis=1)[None, :]

    xe = jax.lax.dot_general(x, emb, (((1,), (1,)), ((), ())),
                             precision=jax.lax.Precision.DEFAULT)
    xnorm = jnp.sum(x * x, axis=1, keepdims=True)
    dist = (xnorm + enorm_ref[...]) - 2.0 * xe
    minval = jnp.min(dist, axis=1, keepdims=True)
    iota = jax.lax.broadcasted_iota(jnp.int32, dist.shape, 1)
    idxc = jnp.min(jnp.where(dist == minval, iota, _K), axis=1, keepdims=True)
    idx_ref[...] = jnp.reshape(idxc, (_BLK,))
    s1 = jnp.sum(minval)

    @pl.when(i == 0)
    def _():
        s1_ref[0] = 0.0

    s1_ref[0] += s1


def _sc_gather_body(emb_hbm, idx_hbm, y_hbm, q_hbm, s2_hbm,
                    idx_v, rows_v, y_v, acc_v, tab_v, sg0, sg1, sy0, sy1):
    c = lax.axis_index("c")
    s = lax.axis_index("s")
    wid = s * _NC + c
    base = wid * _BPW

    @pl.when(s == 0)
    def _():
        pltpu.sync_copy(emb_hbm, tab_v)

    plsc.subcore_barrier()
    pltpu.sync_copy(idx_hbm.at[pl.ds(base, _BPW)], idx_v)
    sgs = (sg0, sg1)
    sys_ = (sy0, sy1)

    def g_copy(ci, b):
        return pltpu.make_async_copy(
            tab_v.at[idx_v.at[pl.ds(ci * _CH, _CH)]], rows_v.at[b], sgs[b])

    def y_copy(ci, b):
        return pltpu.make_async_copy(
            y_hbm.at[pl.ds(base + ci * _CH, _CH)], y_v.at[b], sys_[b])

    g_copy(0, 0).start()
    y_copy(0, 0).start()
    g_copy(1, 1).start()
    y_copy(1, 1).start()
    z = jnp.zeros((_L,), jnp.float32)
    acc4 = (z, z, z, z)
    for ci in range(_NCH):
        b = ci % 2
        g_copy(ci, b).wait()
        y_copy(ci, b).wait()
        rv = rows_v.at[b]
        yv = y_v.at[b]

        def row(r, accs, rv=rv, yv=yv):
            out = []
            for j in range(_D // _L):
                qd = rv[r, pl.ds(j * _L, _L)]
                yd = yv[r, pl.ds(j * _L, _L)]
                d = yd - qd
                out.append(accs[j] + d * d)
            return tuple(out)

        acc4 = lax.fori_loop(0, _CH, row, acc4, unroll=4)
        pltpu.sync_copy(rows_v.at[b], q_hbm.at[pl.ds(base + ci * _CH, _CH)])
        if ci + 2 < _NCH:
            g_copy(ci + 2, b).start()
            y_copy(ci + 2, b).start()
    acc_v[...] = (acc4[0] + acc4[1]) + (acc4[2] + acc4[3])
    pltpu.sync_copy(acc_v, s2_hbm.at[wid])


def kernel(x, y, embeddings):
    idx, s1 = pl.pallas_call(
        _tc_argmin_block,
        grid=(_NB,),
        in_specs=[
            pl.BlockSpec((_BLK, _D), lambda i: (i, 0)),
            pl.BlockSpec((_K, _D), lambda i: (0, 0)),
        ],
        out_specs=[
            pl.BlockSpec((_BLK,), lambda i: (i,)),
            pl.BlockSpec(block_shape=(1,), index_map=lambda i: (0,),
                         memory_space=pltpu.SMEM),
        ],
        out_shape=[
            jax.ShapeDtypeStruct((_N,), jnp.int32),
            jax.ShapeDtypeStruct((1,), jnp.float32),
        ],
        scratch_shapes=[pltpu.VMEM((1, _K), jnp.float32)],
    )(x, embeddings)

    sc = pl.kernel(
        _sc_gather_body,
        out_type=[
            jax.ShapeDtypeStruct((_N, _D), jnp.float32),
            jax.ShapeDtypeStruct((_NW, _L), jnp.float32),
        ],
        mesh=plsc.VectorSubcoreMesh(core_axis_name="c", subcore_axis_name="s"),
        scratch_types=[
            pltpu.VMEM((_BPW,), jnp.int32),
            pltpu.VMEM((2, _CH, _D), jnp.float32),
            pltpu.VMEM((2, _CH, _D), jnp.float32),
            pltpu.VMEM((_L,), jnp.float32),
            pltpu.VMEM_SHARED((_K, _D), jnp.float32),
            pltpu.SemaphoreType.DMA,
            pltpu.SemaphoreType.DMA,
            pltpu.SemaphoreType.DMA,
            pltpu.SemaphoreType.DMA,
        ],
        compiler_params=pltpu.CompilerParams(use_tc_tiling_on_sc=False),
    )
    q, s2p = sc(embeddings, idx, y)
    loss = 1.25 * (s1[0] + jnp.sum(s2p)) / (_N * _D)
    return q, loss


# speedup vs baseline: 1.0071x; 1.0071x over previous
import functools

import jax
import jax.numpy as jnp
from jax import lax
from jax.experimental import pallas as pl
from jax.experimental.pallas import tpu as pltpu
from jax.experimental.pallas import tpu_sc as plsc

_N, _D, _K = 262144, 64, 512
_BLK = 2048
_NB = _N // _BLK

_NC, _NS, _L = 2, 16, 16
_NW = _NC * _NS
_BPW = _N // _NW
_CH = 256
_NCH = _BPW // _CH


def _tc_argmin_block(x_ref, emb_ref, idx_ref, s1_ref, enorm_ref):
    i = pl.program_id(0)
    x = x_ref[...]
    emb = emb_ref[...]

    @pl.when(i == 0)
    def _():
        enorm_ref[...] = jnp.sum(emb * emb, axis=1)[None, :]

    xe = jax.lax.dot_general(x, emb, (((1,), (1,)), ((), ())),
                             precision=jax.lax.Precision.DEFAULT)
    xnorm = jnp.sum(x * x, axis=1, keepdims=True)
    dist = (xnorm + enorm_ref[...]) - 2.0 * xe
    minval = jnp.min(dist, axis=1, keepdims=True)
    iota = jax.lax.broadcasted_iota(jnp.int32, dist.shape, 1)
    idxc = jnp.min(jnp.where(dist == minval, iota, _K), axis=1, keepdims=True)
    idx_ref[...] = idxc
    s1 = jnp.sum(minval)

    @pl.when(i == 0)
    def _():
        s1_ref[0] = 0.0

    s1_ref[0] += s1


def _sc_gather_body(emb_hbm, idx_hbm, y_hbm, q_hbm, s2_hbm,
                    idx_v, rows_v, y_v, acc_v, tab_v, sg0, sg1, sy0, sy1,
                    seg_off=0, bpw=None, nch=None):
    _BPW = bpw if bpw is not None else _N // _NW
    _NCH = nch if nch is not None else _BPW // _CH
    c = lax.axis_index("c")
    s = lax.axis_index("s")
    wid = s * _NC + c
    base = wid * _BPW
    ybase = seg_off + base

    @pl.when(s == 0)
    def _():
        pltpu.sync_copy(emb_hbm, tab_v)

    plsc.subcore_barrier()
    pltpu.sync_copy(idx_hbm.at[pl.ds(base, _BPW)], idx_v)
    sgs = (sg0, sg1)
    sys_ = (sy0, sy1)

    def g_copy(ci, b):
        return pltpu.make_async_copy(
            tab_v.at[idx_v.at[pl.ds(ci * _CH, _CH)]], rows_v.at[b], sgs[b])

    def y_copy(ci, b):
        return pltpu.make_async_copy(
            y_hbm.at[pl.ds(ybase + ci * _CH, _CH)], y_v.at[b], sys_[b])

    g_copy(0, 0).start()
    y_copy(0, 0).start()
    g_copy(1, 1).start()
    y_copy(1, 1).start()
    z = jnp.zeros((_L,), jnp.float32)
    acc4 = (z, z, z, z)
    for ci in range(_NCH):
        b = ci % 2
        g_copy(ci, b).wait()
        y_copy(ci, b).wait()
        rv = rows_v.at[b]
        yv = y_v.at[b]

        def row(r, accs, rv=rv, yv=yv):
            out = []
            for j in range(_D // _L):
                qd = rv[r, pl.ds(j * _L, _L)]
                yd = yv[r, pl.ds(j * _L, _L)]
                d = yd - qd
                out.append(accs[j] + d * d)
            return tuple(out)

        acc4 = lax.fori_loop(0, _CH, row, acc4, unroll=4)
        pltpu.sync_copy(rows_v.at[b], q_hbm.at[pl.ds(base + ci * _CH, _CH)])
        if ci + 2 < _NCH:
            g_copy(ci + 2, b).start()
            y_copy(ci + 2, b).start()
    acc_v[...] = (acc4[0] + acc4[1]) + (acc4[2] + acc4[3])
    pltpu.sync_copy(acc_v, s2_hbm.at[wid])


_S = 2
_NSEG = _N // _S
_NBS = _NSEG // _BLK
_BPWS = _NSEG // _NW
_NCHS = _BPWS // _CH


def _tc_call(x, embeddings, seg):
    return pl.pallas_call(
        _tc_argmin_block,
        grid=(_NBS,),
        in_specs=[
            pl.BlockSpec((_BLK, _D), lambda i, seg=seg: (i + seg * _NBS, 0)),
            pl.BlockSpec((_K, _D), lambda i: (0, 0)),
        ],
        out_specs=[
            pl.BlockSpec((_BLK, 1), lambda i: (i, 0)),
            pl.BlockSpec(block_shape=(1,), index_map=lambda i: (0,),
                         memory_space=pltpu.SMEM),
        ],
        out_shape=[
            jax.ShapeDtypeStruct((_NSEG, 1), jnp.int32),
            jax.ShapeDtypeStruct((1,), jnp.float32),
        ],
        scratch_shapes=[pltpu.VMEM((1, _K), jnp.float32)],
    )(x, embeddings)


def _sc_call(embeddings, idx, y, seg):
    body = functools.partial(_sc_gather_body, seg_off=seg * _NSEG,
                             bpw=_BPWS, nch=_NCHS)
    sc = pl.kernel(
        body,
        out_type=[
            jax.ShapeDtypeStruct((_NSEG, _D), jnp.float32),
            jax.ShapeDtypeStruct((_NW, _L), jnp.float32),
        ],
        mesh=plsc.VectorSubcoreMesh(core_axis_name="c", subcore_axis_name="s"),
        scratch_types=[
            pltpu.VMEM((_BPWS,), jnp.int32),
            pltpu.VMEM((2, _CH, _D), jnp.float32),
            pltpu.VMEM((2, _CH, _D), jnp.float32),
            pltpu.VMEM((_L,), jnp.float32),
            pltpu.VMEM_SHARED((_K, _D), jnp.float32),
            pltpu.SemaphoreType.DMA,
            pltpu.SemaphoreType.DMA,
            pltpu.SemaphoreType.DMA,
            pltpu.SemaphoreType.DMA,
        ],
        compiler_params=pltpu.CompilerParams(use_tc_tiling_on_sc=False),
    )
    return sc(embeddings, idx, y)


def kernel(x, y, embeddings):
    qs, s1s, s2s = [], [], []
    for seg in range(_S):
        idx2d, s1 = _tc_call(x, embeddings, seg)
        q, s2p = _sc_call(embeddings, idx2d.reshape(_NSEG), y, seg)
        qs.append(q)
        s1s.append(s1[0])
        s2s.append(jnp.sum(s2p))
    q = jnp.concatenate(qs, axis=0)
    loss = 1.25 * (sum(s1s) + sum(s2s)) / (_N * _D)
    return q, loss


# speedup vs baseline: 1.0789x; 1.0713x over previous
import functools

import jax
import jax.numpy as jnp
from jax import lax
from jax.experimental import pallas as pl
from jax.experimental.pallas import tpu as pltpu
from jax.experimental.pallas import tpu_sc as plsc

_N, _D, _K = 262144, 64, 512
_BLK = 4096
_NB = _N // _BLK

_NC, _NS, _L = 2, 16, 16
_NW = _NC * _NS
_BPW = _N // _NW
_CH = 256
_NCH = _BPW // _CH


def _tc_argmin_block(x_ref, emb_ref, idx_ref, s1_ref, enorm_ref):
    i = pl.program_id(0)
    x = x_ref[...]
    emb = emb_ref[...]

    @pl.when(i == 0)
    def _():
        enorm_ref[...] = jnp.sum(emb * emb, axis=1)[None, :]

    xe = jax.lax.dot_general(x, emb, (((1,), (1,)), ((), ())),
                             precision=jax.lax.Precision.DEFAULT)
    xnorm = jnp.sum(x * x, axis=1, keepdims=True)
    dist = (xnorm + enorm_ref[...]) - 2.0 * xe
    minval = jnp.min(dist, axis=1, keepdims=True)
    iota = jax.lax.broadcasted_iota(jnp.int32, dist.shape, 1)
    idxc = jnp.min(jnp.where(dist == minval, iota, _K), axis=1, keepdims=True)
    idx_ref[...] = idxc
    s1 = jnp.sum(minval)

    @pl.when(i == 0)
    def _():
        s1_ref[0] = 0.0

    s1_ref[0] += s1


def _sc_gather_body(emb_hbm, idx_hbm, y_hbm, q_hbm, s2_hbm,
                    idx_v, rows_v, y_v, acc_v, tab_v, sg0, sg1, sy0, sy1):
    c = lax.axis_index("c")
    s = lax.axis_index("s")
    wid = s * _NC + c
    base = wid * _BPW

    @pl.when(s == 0)
    def _():
        pltpu.sync_copy(emb_hbm, tab_v)

    plsc.subcore_barrier()
    pltpu.sync_copy(idx_hbm.at[pl.ds(base, _BPW)], idx_v)
    sgs = (sg0, sg1)
    sys_ = (sy0, sy1)

    def g_copy(ci, b):
        return pltpu.make_async_copy(
            tab_v.at[idx_v.at[pl.ds(ci * _CH, _CH)]], rows_v.at[b], sgs[b])

    def y_copy(ci, b):
        return pltpu.make_async_copy(
            y_hbm.at[pl.ds(base + ci * _CH, _CH)], y_v.at[b], sys_[b])

    g_copy(0, 0).start()
    y_copy(0, 0).start()
    g_copy(1, 1).start()
    y_copy(1, 1).start()
    z = jnp.zeros((_L,), jnp.float32)
    acc4 = (z, z, z, z)
    for ci in range(_NCH):
        b = ci % 2
        g_copy(ci, b).wait()
        y_copy(ci, b).wait()
        rv = rows_v.at[b]
        yv = y_v.at[b]

        def row(r, accs, rv=rv, yv=yv):
            out = []
            for j in range(_D // _L):
                qd = rv[r, pl.ds(j * _L, _L)]
                yd = yv[r, pl.ds(j * _L, _L)]
                d = yd - qd
                out.append(accs[j] + d * d)
            return tuple(out)

        acc4 = lax.fori_loop(0, _CH, row, acc4, unroll=4)
        pltpu.sync_copy(rows_v.at[b], q_hbm.at[pl.ds(base + ci * _CH, _CH)])
        if ci + 2 < _NCH:
            g_copy(ci + 2, b).start()
            y_copy(ci + 2, b).start()
    acc_v[...] = (acc4[0] + acc4[1]) + (acc4[2] + acc4[3])
    pltpu.sync_copy(acc_v, s2_hbm.at[wid])


def kernel(x, y, embeddings):
    idx2d, s1 = pl.pallas_call(
        _tc_argmin_block,
        grid=(_NB,),
        in_specs=[
            pl.BlockSpec((_BLK, _D), lambda i: (i, 0)),
            pl.BlockSpec((_K, _D), lambda i: (0, 0)),
        ],
        out_specs=[
            pl.BlockSpec((_BLK, 1), lambda i: (i, 0)),
            pl.BlockSpec(block_shape=(1,), index_map=lambda i: (0,),
                         memory_space=pltpu.SMEM),
        ],
        out_shape=[
            jax.ShapeDtypeStruct((_N, 1), jnp.int32),
            jax.ShapeDtypeStruct((1,), jnp.float32),
        ],
        scratch_shapes=[pltpu.VMEM((1, _K), jnp.float32)],
    )(x, embeddings)

    idx = idx2d.reshape(_N)
    sc = pl.kernel(
        _sc_gather_body,
        out_type=[
            jax.ShapeDtypeStruct((_N, _D), jnp.float32),
            jax.ShapeDtypeStruct((_NW, _L), jnp.float32),
        ],
        mesh=plsc.VectorSubcoreMesh(core_axis_name="c", subcore_axis_name="s"),
        scratch_types=[
            pltpu.VMEM((_BPW,), jnp.int32),
            pltpu.VMEM((2, _CH, _D), jnp.float32),
            pltpu.VMEM((2, _CH, _D), jnp.float32),
            pltpu.VMEM((_L,), jnp.float32),
            pltpu.VMEM_SHARED((_K, _D), jnp.float32),
            pltpu.SemaphoreType.DMA,
            pltpu.SemaphoreType.DMA,
            pltpu.SemaphoreType.DMA,
            pltpu.SemaphoreType.DMA,
        ],
        compiler_params=pltpu.CompilerParams(use_tc_tiling_on_sc=False),
    )
    q, s2p = sc(embeddings, idx, y)
    loss = 1.25 * (s1[0] + jnp.sum(s2p)) / (_N * _D)
    return q, loss


# speedup vs baseline: 1.0907x; 1.0109x over previous
import functools

import jax
import jax.numpy as jnp
from jax import lax
from jax.experimental import pallas as pl
from jax.experimental.pallas import tpu as pltpu
from jax.experimental.pallas import tpu_sc as plsc

_N, _D, _K = 262144, 64, 512
_BLK = 8192
_NB = _N // _BLK

_NC, _NS, _L = 2, 16, 16
_NW = _NC * _NS
_BPW = _N // _NW
_CH = 256
_NCH = _BPW // _CH


def _tc_argmin_block(x_ref, emb_ref, idx_ref, s1_ref, enorm_ref):
    i = pl.program_id(0)
    x = x_ref[...]
    emb = emb_ref[...]

    @pl.when(i == 0)
    def _():
        enorm_ref[...] = jnp.sum(emb * emb, axis=1)[None, :]

    xe = jax.lax.dot_general(x, emb, (((1,), (1,)), ((), ())),
                             precision=jax.lax.Precision.DEFAULT)
    xnorm = jnp.sum(x * x, axis=1, keepdims=True)
    dist = (xnorm + enorm_ref[...]) - 2.0 * xe
    minval = jnp.min(dist, axis=1, keepdims=True)
    iota = jax.lax.broadcasted_iota(jnp.int32, dist.shape, 1)
    idxc = jnp.min(jnp.where(dist == minval, iota, _K), axis=1, keepdims=True)
    idx_ref[...] = idxc
    s1 = jnp.sum(minval)

    @pl.when(i == 0)
    def _():
        s1_ref[0] = 0.0

    s1_ref[0] += s1


def _sc_gather_body(emb_hbm, idx_hbm, y_hbm, q_hbm, s2_hbm,
                    idx_v, rows_v, y_v, acc_v, tab_v, sg0, sg1, sy0, sy1):
    c = lax.axis_index("c")
    s = lax.axis_index("s")
    wid = s * _NC + c
    base = wid * _BPW

    @pl.when(s == 0)
    def _():
        pltpu.sync_copy(emb_hbm, tab_v)

    plsc.subcore_barrier()
    pltpu.sync_copy(idx_hbm.at[pl.ds(base, _BPW)], idx_v)
    sgs = (sg0, sg1)
    sys_ = (sy0, sy1)

    def g_copy(ci, b):
        return pltpu.make_async_copy(
            tab_v.at[idx_v.at[pl.ds(ci * _CH, _CH)]], rows_v.at[b], sgs[b])

    def y_copy(ci, b):
        return pltpu.make_async_copy(
            y_hbm.at[pl.ds(base + ci * _CH, _CH)], y_v.at[b], sys_[b])

    g_copy(0, 0).start()
    y_copy(0, 0).start()
    g_copy(1, 1).start()
    y_copy(1, 1).start()
    z = jnp.zeros((_L,), jnp.float32)
    acc4 = (z, z, z, z)
    for ci in range(_NCH):
        b = ci % 2
        g_copy(ci, b).wait()
        y_copy(ci, b).wait()
        rv = rows_v.at[b]
        yv = y_v.at[b]

        def row(r, accs, rv=rv, yv=yv):
            out = []
            for j in range(_D // _L):
                qd = rv[r, pl.ds(j * _L, _L)]
                yd = yv[r, pl.ds(j * _L, _L)]
                d = yd - qd
                out.append(accs[j] + d * d)
            return tuple(out)

        acc4 = lax.fori_loop(0, _CH, row, acc4, unroll=4)
        pltpu.sync_copy(rows_v.at[b], q_hbm.at[pl.ds(base + ci * _CH, _CH)])
        if ci + 2 < _NCH:
            g_copy(ci + 2, b).start()
            y_copy(ci + 2, b).start()
    acc_v[...] = (acc4[0] + acc4[1]) + (acc4[2] + acc4[3])
    pltpu.sync_copy(acc_v, s2_hbm.at[wid])


def kernel(x, y, embeddings):
    idx2d, s1 = pl.pallas_call(
        _tc_argmin_block,
        grid=(_NB,),
        in_specs=[
            pl.BlockSpec((_BLK, _D), lambda i: (i, 0)),
            pl.BlockSpec((_K, _D), lambda i: (0, 0)),
        ],
        out_specs=[
            pl.BlockSpec((_BLK, 1), lambda i: (i, 0)),
            pl.BlockSpec(block_shape=(1,), index_map=lambda i: (0,),
                         memory_space=pltpu.SMEM),
        ],
        out_shape=[
            jax.ShapeDtypeStruct((_N, 1), jnp.int32),
            jax.ShapeDtypeStruct((1,), jnp.float32),
        ],
        scratch_shapes=[pltpu.VMEM((1, _K), jnp.float32)],
    )(x, embeddings)

    idx = idx2d.reshape(_N)
    sc = pl.kernel(
        _sc_gather_body,
        out_type=[
            jax.ShapeDtypeStruct((_N, _D), jnp.float32),
            jax.ShapeDtypeStruct((_NW, _L), jnp.float32),
        ],
        mesh=plsc.VectorSubcoreMesh(core_axis_name="c", subcore_axis_name="s"),
        scratch_types=[
            pltpu.VMEM((_BPW,), jnp.int32),
            pltpu.VMEM((2, _CH, _D), jnp.float32),
            pltpu.VMEM((2, _CH, _D), jnp.float32),
            pltpu.VMEM((_L,), jnp.float32),
            pltpu.VMEM_SHARED((_K, _D), jnp.float32),
            pltpu.SemaphoreType.DMA,
            pltpu.SemaphoreType.DMA,
            pltpu.SemaphoreType.DMA,
            pltpu.SemaphoreType.DMA,
        ],
        compiler_params=pltpu.CompilerParams(use_tc_tiling_on_sc=False),
    )
    q, s2p = sc(embeddings, idx, y)
    loss = 1.25 * (s1[0] + jnp.sum(s2p)) / (_N * _D)
    return q, loss
